# baseline (speedup 1.0000x reference)
import jax
import jax.numpy as jnp
from jax import lax
from jax.experimental import pallas as pl
from jax.experimental.pallas import tpu as pltpu
from jax.experimental.pallas import tpu_sc as plsc

_NF = 26
_FD = 16
_NPAIR = (_NF * (_NF - 1)) // 2
_ROW_IN = _NF * _FD
_ROW_OUT = _NPAIR * 3 * _FD
_BATCH = 4096
_NW = 32
_LANES = _BATCH // _NW
_NSUB = _LANES // 16
_G = 5
_NG = _NPAIR // _G
_GROWS = _G * 3 * _FD


def _advance(i, j):
    nj = j + 1
    wrap = nj >= _NF
    ni = lax.select(wrap, i + 1, i)
    nj = lax.select(wrap, ni + 1, nj)
    return ni, nj


def _pin_body(xt_hbm, out_hbm, xblk, obuf, sem0, sem1):
    wid = lax.axis_index("s") * 2 + lax.axis_index("c")
    lane0 = wid * _LANES

    pltpu.sync_copy(xt_hbm.at[:, pl.ds(lane0, _LANES)], xblk)

    sems = (sem0, sem1)

    def group(g, carry):
        i0, j0 = carry
        slot = lax.rem(g, 2)

        ijs = [(i0, j0)]
        for _ in range(_G - 1):
            ijs.append(_advance(*ijs[-1]))
        nxt = _advance(*ijs[-1])

        for k in range(2):
            @pl.when((slot == k) & (g >= 2))
            def _drain(k=k):
                pltpu.make_async_copy(
                    obuf.at[k],
                    out_hbm.at[pl.ds(0, _GROWS), pl.ds(lane0, _LANES)],
                    sems[k]).wait()

        for k in range(_G):
            ik, jk = ijs[k]
            ir = _FD * ik
            jr = _FD * jk
            r0 = 3 * _FD * k
            for c in range(_FD):
                for u in range(_NSUB):
                    sl = pl.ds(16 * u, 16)
                    av = xblk[ir + c, sl]
                    bv = xblk[jr + c, sl]
                    obuf[slot, r0 + c, sl] = av
                    obuf[slot, r0 + _FD + c, sl] = bv
                    obuf[slot, r0 + 2 * _FD + c, sl] = av * bv

        for k in range(2):
            @pl.when(slot == k)
            def _issue(k=k):
                pltpu.async_copy(
                    obuf.at[k],
                    out_hbm.at[pl.ds(_GROWS * g, _GROWS),
                               pl.ds(lane0, _LANES)],
                    sems[k])
        return nxt

    lax.fori_loop(0, _NG, group, (jnp.int32(0), jnp.int32(1)))

    for k in range(2):
        pltpu.make_async_copy(
            obuf.at[k],
            out_hbm.at[pl.ds(0, _GROWS), pl.ds(lane0, _LANES)],
            sems[k]).wait()


@jax.jit
def kernel(x):
    xt = x.transpose(1, 2, 0).reshape(_ROW_IN, _BATCH)
    run = pl.kernel(
        _pin_body,
        out_type=jax.ShapeDtypeStruct((_ROW_OUT, _BATCH), jnp.float32),
        scratch_types=[
            pltpu.VMEM((_ROW_IN, _LANES), jnp.float32),
            pltpu.VMEM((2, _GROWS, _LANES), jnp.float32),
            pltpu.SemaphoreType.DMA,
            pltpu.SemaphoreType.DMA,
        ],
        mesh=plsc.VectorSubcoreMesh(core_axis_name="c", subcore_axis_name="s"),
    )
    out_t = run(xt)
    return out_t.reshape(_NPAIR, 3 * _FD, _BATCH).transpose(2, 0, 1)
